# initial kernel scaffold (unmeasured)
import jax
import jax.numpy as jnp
from jax import lax
from jax.experimental import pallas as pl
from jax.experimental.pallas import tpu as pltpu


def kernel(
    x,
):
    def body(*refs):
        pass

    out_shape = jax.ShapeDtypeStruct(..., jnp.float32)
    return pl.pallas_call(body, out_shape=out_shape)(...)


# baseline (device time: 103538 ns/iter reference)
import jax
import jax.numpy as jnp
from jax import lax
from jax.experimental import pallas as pl
from jax.experimental.pallas import tpu as pltpu


def kernel(x):
    m, n = x.shape

    def body(x_ref, out_ref, my_ref, rx_ref,
             send_sem1, recv_sem1, send_sem2, recv_sem2):
        my_x = lax.axis_index("x")
        my_y = lax.axis_index("y")
        other_x = 1 - my_x
        other_y = 1 - my_y

        barrier_sem = pltpu.get_barrier_semaphore()
        pl.semaphore_signal(barrier_sem, inc=1, device_id=(other_x, my_y),
                            device_id_type=pl.DeviceIdType.MESH)
        pl.semaphore_signal(barrier_sem, inc=1, device_id=(my_x, other_y),
                            device_id_type=pl.DeviceIdType.MESH)
        pl.semaphore_wait(barrier_sem, 2)

        my_ref[...] = x_ref[...].astype(jnp.bfloat16)

        rdma1 = pltpu.make_async_remote_copy(
            src_ref=my_ref, dst_ref=rx_ref,
            send_sem=send_sem1, recv_sem=recv_sem1,
            device_id=(other_x, my_y), device_id_type=pl.DeviceIdType.MESH,
        )
        rdma1.start()
        rdma1.wait()

        col = pl.ds(my_y * n, n)
        out_ref[:, col] = my_ref[...] + rx_ref[...]

        rdma2 = pltpu.make_async_remote_copy(
            src_ref=out_ref.at[:, col], dst_ref=out_ref.at[:, col],
            send_sem=send_sem2, recv_sem=recv_sem2,
            device_id=(my_x, other_y), device_id_type=pl.DeviceIdType.MESH,
        )
        rdma2.start()
        rdma2.wait()

    return pl.pallas_call(
        body,
        out_shape=jax.ShapeDtypeStruct((m, 2 * n), jnp.bfloat16),
        in_specs=[pl.BlockSpec(memory_space=pltpu.VMEM)],
        out_specs=pl.BlockSpec(memory_space=pltpu.VMEM),
        scratch_shapes=[
            pltpu.VMEM((m, n), jnp.bfloat16),
            pltpu.VMEM((m, n), jnp.bfloat16),
            pltpu.SemaphoreType.DMA,
            pltpu.SemaphoreType.DMA,
            pltpu.SemaphoreType.DMA,
            pltpu.SemaphoreType.DMA,
        ],
        compiler_params=pltpu.CompilerParams(collective_id=0),
    )(x)


# device time: 63747 ns/iter; 1.6242x vs baseline; 1.6242x over previous
import jax
import jax.numpy as jnp
from jax import lax
from jax.experimental import pallas as pl
from jax.experimental.pallas import tpu as pltpu

NCHUNK = 8


def kernel(x):
    m, n = x.shape
    mc = m // NCHUNK

    def body(x_ref, out_ref, my_ref, rx_ref,
             send_sem1, recv_sem1, send_sem2, recv_sem2):
        my_x = lax.axis_index("x")
        my_y = lax.axis_index("y")
        other_x = 1 - my_x
        other_y = 1 - my_y
        col = pl.ds(my_y * n, n)

        def rdma1(k):
            rows = pl.ds(k * mc, mc)
            return pltpu.make_async_remote_copy(
                src_ref=my_ref.at[rows, :], dst_ref=rx_ref.at[rows, :],
                send_sem=send_sem1.at[k], recv_sem=recv_sem1.at[k],
                device_id=(other_x, my_y), device_id_type=pl.DeviceIdType.MESH,
            )

        def rdma2(k):
            rows = pl.ds(k * mc, mc)
            return pltpu.make_async_remote_copy(
                src_ref=out_ref.at[rows, col], dst_ref=out_ref.at[rows, col],
                send_sem=send_sem2.at[k], recv_sem=recv_sem2.at[k],
                device_id=(my_x, other_y), device_id_type=pl.DeviceIdType.MESH,
            )

        barrier_sem = pltpu.get_barrier_semaphore()
        pl.semaphore_signal(barrier_sem, inc=1, device_id=(other_x, my_y),
                            device_id_type=pl.DeviceIdType.MESH)
        pl.semaphore_signal(barrier_sem, inc=1, device_id=(my_x, other_y),
                            device_id_type=pl.DeviceIdType.MESH)
        pl.semaphore_wait(barrier_sem, 2)

        for k in range(NCHUNK):
            rows = pl.ds(k * mc, mc)
            my_ref[rows, :] = x_ref[rows, :].astype(jnp.bfloat16)
            rdma1(k).start()

        for k in range(NCHUNK):
            rows = pl.ds(k * mc, mc)
            rdma1(k).wait_recv()
            out_ref[rows, col] = my_ref[rows, :] + rx_ref[rows, :]
            rdma2(k).start()

        for k in range(NCHUNK):
            rdma1(k).wait_send()
            rdma2(k).wait()

    return pl.pallas_call(
        body,
        out_shape=jax.ShapeDtypeStruct((m, 2 * n), jnp.bfloat16),
        in_specs=[pl.BlockSpec(memory_space=pltpu.VMEM)],
        out_specs=pl.BlockSpec(memory_space=pltpu.VMEM),
        scratch_shapes=[
            pltpu.VMEM((m, n), jnp.bfloat16),
            pltpu.VMEM((m, n), jnp.bfloat16),
            pltpu.SemaphoreType.DMA((NCHUNK,)),
            pltpu.SemaphoreType.DMA((NCHUNK,)),
            pltpu.SemaphoreType.DMA((NCHUNK,)),
            pltpu.SemaphoreType.DMA((NCHUNK,)),
        ],
        compiler_params=pltpu.CompilerParams(collective_id=0),
    )(x)


# device time: 61185 ns/iter; 1.6922x vs baseline; 1.0419x over previous
import jax
import jax.numpy as jnp
from jax import lax
from jax.experimental import pallas as pl
from jax.experimental.pallas import tpu as pltpu

NCHUNK = 16


def kernel(x):
    m, n = x.shape
    mc = m // NCHUNK

    def body(x_ref, out_ref, my_ref, rx_ref,
             send_sem1, recv_sem1, send_sem2, recv_sem2):
        my_x = lax.axis_index("x")
        my_y = lax.axis_index("y")
        other_x = 1 - my_x
        other_y = 1 - my_y
        col = pl.ds(my_y * n, n)

        def rdma1(k):
            rows = pl.ds(k * mc, mc)
            return pltpu.make_async_remote_copy(
                src_ref=my_ref.at[rows, :], dst_ref=rx_ref.at[rows, :],
                send_sem=send_sem1.at[k], recv_sem=recv_sem1.at[k],
                device_id=(other_x, my_y), device_id_type=pl.DeviceIdType.MESH,
            )

        def rdma2(k):
            rows = pl.ds(k * mc, mc)
            return pltpu.make_async_remote_copy(
                src_ref=out_ref.at[rows, col], dst_ref=out_ref.at[rows, col],
                send_sem=send_sem2.at[k], recv_sem=recv_sem2.at[k],
                device_id=(my_x, other_y), device_id_type=pl.DeviceIdType.MESH,
            )

        barrier_sem = pltpu.get_barrier_semaphore()
        pl.semaphore_signal(barrier_sem, inc=1, device_id=(other_x, my_y),
                            device_id_type=pl.DeviceIdType.MESH)
        pl.semaphore_signal(barrier_sem, inc=1, device_id=(my_x, other_y),
                            device_id_type=pl.DeviceIdType.MESH)
        pl.semaphore_wait(barrier_sem, 2)

        for k in range(NCHUNK):
            rows = pl.ds(k * mc, mc)
            my_ref[rows, :] = x_ref[rows, :].astype(jnp.bfloat16)
            rdma1(k).start()

        for k in range(NCHUNK):
            rows = pl.ds(k * mc, mc)
            rdma1(k).wait_recv()
            out_ref[rows, col] = my_ref[rows, :] + rx_ref[rows, :]
            rdma2(k).start()

        for k in range(NCHUNK):
            rdma1(k).wait_send()
            rdma2(k).wait()

    return pl.pallas_call(
        body,
        out_shape=jax.ShapeDtypeStruct((m, 2 * n), jnp.bfloat16),
        in_specs=[pl.BlockSpec(memory_space=pltpu.VMEM)],
        out_specs=pl.BlockSpec(memory_space=pltpu.VMEM),
        scratch_shapes=[
            pltpu.VMEM((m, n), jnp.bfloat16),
            pltpu.VMEM((m, n), jnp.bfloat16),
            pltpu.SemaphoreType.DMA((NCHUNK,)),
            pltpu.SemaphoreType.DMA((NCHUNK,)),
            pltpu.SemaphoreType.DMA((NCHUNK,)),
            pltpu.SemaphoreType.DMA((NCHUNK,)),
        ],
        compiler_params=pltpu.CompilerParams(collective_id=0),
    )(x)


# device time: 60269 ns/iter; 1.7179x vs baseline; 1.0152x over previous
import jax
import jax.numpy as jnp
from jax import lax
from jax.experimental import pallas as pl
from jax.experimental.pallas import tpu as pltpu

NCHUNK = 32


def kernel(x):
    m, n = x.shape
    mc = m // NCHUNK

    def body(x_ref, out_ref, my_ref, rx_ref,
             send_sem1, recv_sem1, send_sem2, recv_sem2):
        my_x = lax.axis_index("x")
        my_y = lax.axis_index("y")
        other_x = 1 - my_x
        other_y = 1 - my_y
        col = pl.ds(my_y * n, n)

        def rdma1(k):
            rows = pl.ds(k * mc, mc)
            return pltpu.make_async_remote_copy(
                src_ref=my_ref.at[rows, :], dst_ref=rx_ref.at[rows, :],
                send_sem=send_sem1.at[k], recv_sem=recv_sem1.at[k],
                device_id=(other_x, my_y), device_id_type=pl.DeviceIdType.MESH,
            )

        def rdma2(k):
            rows = pl.ds(k * mc, mc)
            return pltpu.make_async_remote_copy(
                src_ref=out_ref.at[rows, col], dst_ref=out_ref.at[rows, col],
                send_sem=send_sem2.at[k], recv_sem=recv_sem2.at[k],
                device_id=(my_x, other_y), device_id_type=pl.DeviceIdType.MESH,
            )

        barrier_sem = pltpu.get_barrier_semaphore()
        pl.semaphore_signal(barrier_sem, inc=1, device_id=(other_x, my_y),
                            device_id_type=pl.DeviceIdType.MESH)
        pl.semaphore_signal(barrier_sem, inc=1, device_id=(my_x, other_y),
                            device_id_type=pl.DeviceIdType.MESH)
        pl.semaphore_wait(barrier_sem, 2)

        for k in range(NCHUNK):
            rows = pl.ds(k * mc, mc)
            my_ref[rows, :] = x_ref[rows, :].astype(jnp.bfloat16)
            rdma1(k).start()

        for k in range(NCHUNK):
            rows = pl.ds(k * mc, mc)
            rdma1(k).wait_recv()
            out_ref[rows, col] = my_ref[rows, :] + rx_ref[rows, :]
            rdma2(k).start()

        for k in range(NCHUNK):
            rdma1(k).wait_send()
            rdma2(k).wait()

    return pl.pallas_call(
        body,
        out_shape=jax.ShapeDtypeStruct((m, 2 * n), jnp.bfloat16),
        in_specs=[pl.BlockSpec(memory_space=pltpu.VMEM)],
        out_specs=pl.BlockSpec(memory_space=pltpu.VMEM),
        scratch_shapes=[
            pltpu.VMEM((m, n), jnp.bfloat16),
            pltpu.VMEM((m, n), jnp.bfloat16),
            pltpu.SemaphoreType.DMA((NCHUNK,)),
            pltpu.SemaphoreType.DMA((NCHUNK,)),
            pltpu.SemaphoreType.DMA((NCHUNK,)),
            pltpu.SemaphoreType.DMA((NCHUNK,)),
        ],
        compiler_params=pltpu.CompilerParams(collective_id=0),
    )(x)
